# Initial kernel scaffold; baseline (speedup 1.0000x reference)
import jax
import jax.numpy as jnp
from jax.experimental import pallas as pl


def kernel(x, edge_index, enc_w1, enc_b1, enc_w2, enc_b2, gcn_w1, gcn_b1, gcn_w2, gcn_b2, lp_w1, lp_b1, lp_w2, lp_b2, v_w1, v_b1, v_w2, v_b2):
    raise NotImplementedError("write your pallas kernel here")


# speedup vs baseline: 5.9012x; 5.9012x over previous
import functools
import jax
import jax.numpy as jnp
from jax import lax
from jax.experimental import pallas as pl
from jax.experimental.pallas import tpu as pltpu
from jax.experimental.pallas import tpu_sc as plsc

_N = 10000
_E = 320000
_D = 128
_H = 128

_NC = 2
_NS = 16
_NHALF = _N // _NC
_CH = 128
_NCHUNK = 160
_EPAD = _NS * _NCHUNK * _CH
_APAD = 5120
_RPT = _APAD // _NS
_DGPAD = 10240
_DRPT = _DGPAD // _NS
_R = 1000


def _deg_body(dst_hbm, out_hbm, didx, ones_v, zbuf, deg_sh):
    cid = lax.axis_index("c")
    sid = lax.axis_index("s")
    pltpu.sync_copy(dst_hbm.at[sid], didx)
    for r in range(16):
        zbuf[r, :] = jnp.zeros((16,), jnp.float32)
    for r in range(_CH):
        ones_v[r, :] = jnp.ones((16,), jnp.float32)
    base = sid * _DRPT
    @pl.loop(0, _DRPT // 16)
    def _zero(k):
        pltpu.sync_copy(zbuf, deg_sh.at[pl.ds(base + k * 16, 16)])
    plsc.subcore_barrier()
    @pl.loop(0, _NCHUNK)
    def _acc(j):
        pltpu.sync_copy(ones_v, deg_sh.at[didx.at[j]], add=True)
    plsc.subcore_barrier()
    pltpu.sync_copy(deg_sh.at[pl.ds(base, _DRPT)],
                    out_hbm.at[cid].at[pl.ds(base, _DRPT)])


def _agg_body(g_hbm, src_hbm, dst_hbm, out_hbm,
              sidx, didx, rows0, rows1, zbuf, acc_sh, sem0, sem1):
    cid = lax.axis_index("c")
    sid = lax.axis_index("s")
    pltpu.sync_copy(src_hbm.at[sid], sidx)
    pltpu.sync_copy(dst_hbm.at[sid], didx)

    lo = cid * _NHALF
    @pl.loop(0, _NCHUNK)
    def _remap(j):
        for c in range(_CH // 16):
            d = didx[j, pl.ds(c * 16, 16)] - lo
            ok = (d >= 0) & (d < _NHALF)
            didx[j, pl.ds(c * 16, 16)] = jnp.where(ok, d, _NHALF)

    for r in range(16):
        for c in range(_H // 16):
            zbuf[r, pl.ds(c * 16, 16)] = jnp.zeros((16,), jnp.float32)
    base = sid * _RPT
    @pl.loop(0, _RPT // 16)
    def _zero(k):
        pltpu.sync_copy(zbuf, acc_sh.at[pl.ds(base + k * 16, 16)])
    plsc.subcore_barrier()

    rows = (rows0, rows1)
    sems = (sem0, sem1)
    pltpu.async_copy(g_hbm.at[sidx.at[0]], rows0, sem0)
    pltpu.async_copy(g_hbm.at[sidx.at[1]], rows1, sem1)
    @pl.loop(0, _NCHUNK, step=2)
    def _chunks(j):
        for b in range(2):
            jj = j + b
            pltpu.make_async_copy(g_hbm.at[sidx.at[jj]], rows[b],
                                  sems[b]).wait()
            pltpu.sync_copy(rows[b], acc_sh.at[didx.at[jj]], add=True)
            @pl.when(jj + 2 < _NCHUNK)
            def _next():
                pltpu.async_copy(g_hbm.at[sidx.at[jj + 2]], rows[b], sems[b])
    plsc.subcore_barrier()
    pltpu.sync_copy(acc_sh.at[pl.ds(base, _RPT)],
                    out_hbm.at[cid].at[pl.ds(base, _RPT)])


@functools.lru_cache(maxsize=1)
def _sc_kernels():
    mesh = plsc.VectorSubcoreMesh(core_axis_name="c", subcore_axis_name="s",
                                  num_cores=_NC, num_subcores=_NS)
    deg_call = pl.kernel(
        _deg_body,
        out_type=jax.ShapeDtypeStruct((_NC, _DGPAD, 16), jnp.float32),
        mesh=mesh,
        scratch_types=[
            pltpu.VMEM((_NCHUNK, _CH), jnp.int32),
            pltpu.VMEM((_CH, 16), jnp.float32),
            pltpu.VMEM((16, 16), jnp.float32),
            pltpu.VMEM_SHARED((_DGPAD, 16), jnp.float32),
        ],
    )
    agg_call = pl.kernel(
        _agg_body,
        out_type=jax.ShapeDtypeStruct((_NC, _APAD, _H), jnp.float32),
        mesh=mesh,
        scratch_types=[
            pltpu.VMEM((_NCHUNK, _CH), jnp.int32),
            pltpu.VMEM((_NCHUNK, _CH), jnp.int32),
            pltpu.VMEM((_CH, _H), jnp.float32),
            pltpu.VMEM((_CH, _H), jnp.float32),
            pltpu.VMEM((16, _H), jnp.float32),
            pltpu.VMEM_SHARED((_APAD, _H), jnp.float32),
            pltpu.SemaphoreType.DMA,
            pltpu.SemaphoreType.DMA,
        ],
    )
    return deg_call, agg_call


def _dinv_block(dp):
    return lax.rsqrt(dp[0, :, 0] + 1.0)


def _k1_body(x_ref, w1_ref, b1_ref, w2_ref, b2_ref, gw_ref, dp_ref, g_ref):
    h = jax.nn.relu(jnp.dot(x_ref[...], w1_ref[...],
                            preferred_element_type=jnp.float32) + b1_ref[...])
    h = jax.nn.relu(jnp.dot(h, w2_ref[...],
                            preferred_element_type=jnp.float32) + b2_ref[...])
    dinv = _dinv_block(dp_ref[...])
    g_ref[...] = jnp.dot(h, gw_ref[...],
                         preferred_element_type=jnp.float32) * dinv[:, None]


def _combine(a_ref, g_ref, dp_ref, b_ref):
    dinv = _dinv_block(dp_ref[...])
    tot = a_ref[0] + g_ref[...]
    return jax.nn.relu(tot * dinv[:, None] + b_ref[...]), dinv


def _k2_body(a_ref, g_ref, dp_ref, b_ref, gw_ref, out_ref):
    h, dinv = _combine(a_ref, g_ref, dp_ref, b_ref)
    out_ref[...] = jnp.dot(h, gw_ref[...],
                           preferred_element_type=jnp.float32) * dinv[:, None]


def _k3_body(a_ref, g_ref, dp_ref, b_ref,
             lpw1_ref, lpb1_ref, lpw2_ref, lpb2_ref,
             vw1_ref, vb1_ref, vw2_ref, vb2_ref,
             probs_ref, pool_ref, sv_ref):
    i = pl.program_id(0)
    h, _ = _combine(a_ref, g_ref, dp_ref, b_ref)
    lp = jax.nn.relu(jnp.dot(h, lpw1_ref[...],
                             preferred_element_type=jnp.float32) + lpb1_ref[...])
    logit = jnp.dot(lp, lpw2_ref[...],
                    preferred_element_type=jnp.float32) + lpb2_ref[0, 0]
    probs_ref[...] = jax.nn.sigmoid(logit)
    bsum = jnp.sum(h, axis=0, keepdims=True)

    @pl.when(i == 0)
    def _init():
        pool_ref[...] = bsum
        sv_ref[...] = jnp.zeros_like(sv_ref)

    @pl.when(i > 0)
    def _acc():
        pool_ref[...] = pool_ref[...] + bsum

    @pl.when(i == pl.num_programs(0) - 1)
    def _value():
        pooled = pool_ref[...] * (1.0 / _N)
        v = jax.nn.relu(jnp.dot(pooled, vw1_ref[...],
                                preferred_element_type=jnp.float32)
                        + vb1_ref[...])
        sval = jnp.dot(v, vw2_ref[...],
                       preferred_element_type=jnp.float32) + vb2_ref[0, 0]
        sv_ref[...] = jnp.broadcast_to(sval, sv_ref.shape)


def _row_spec(last):
    return pl.BlockSpec((_R, last), lambda i: (i, 0))


def _full_spec(shape):
    nd = len(shape)
    return pl.BlockSpec(shape, lambda i: (0,) * nd)


_dp_spec = pl.BlockSpec((_NC, _R, 16), lambda i: (0, i, 0))
_acc_spec = pl.BlockSpec((1, _R, _H), lambda i: (i // 5, i % 5, 0))


def kernel(x, edge_index, enc_w1, enc_b1, enc_w2, enc_b2,
           gcn_w1, gcn_b1, gcn_w2, gcn_b2,
           lp_w1, lp_b1, lp_w2, lp_b2,
           v_w1, v_b1, v_w2, v_b2):
    src = edge_index[0]
    dst = edge_index[1]
    pad = _EPAD - _E
    srcp = jnp.concatenate([src, jnp.zeros((pad,), jnp.int32)])
    srcp = srcp.reshape(_NS, _NCHUNK, _CH)
    dstp = jnp.concatenate([dst, jnp.full((pad,), _N, jnp.int32)])
    dstp = dstp.reshape(_NS, _NCHUNK, _CH)

    deg_call, agg_call = _sc_kernels()
    dp = deg_call(dstp)

    grid = _N // _R
    g1 = pl.pallas_call(
        _k1_body,
        grid=(grid,),
        in_specs=[_row_spec(_D), _full_spec((_D, _H)), _full_spec((1, _H)),
                  _full_spec((_H, _H)), _full_spec((1, _H)),
                  _full_spec((_H, _H)), _dp_spec],
        out_specs=_row_spec(_H),
        out_shape=jax.ShapeDtypeStruct((_N, _H), jnp.float32),
    )(x, enc_w1, enc_b1.reshape(1, _H), enc_w2, enc_b2.reshape(1, _H),
      gcn_w1, dp)

    a1 = agg_call(g1, srcp, dstp)
    g2 = pl.pallas_call(
        _k2_body,
        grid=(grid,),
        in_specs=[_acc_spec, _row_spec(_H), _dp_spec, _full_spec((1, _H)),
                  _full_spec((_H, _H))],
        out_specs=_row_spec(_H),
        out_shape=jax.ShapeDtypeStruct((_N, _H), jnp.float32),
    )(a1, g1, dp, gcn_b1.reshape(1, _H), gcn_w2)

    a2 = agg_call(g2, srcp, dstp)
    probs, _, sv = pl.pallas_call(
        _k3_body,
        grid=(grid,),
        in_specs=[_acc_spec, _row_spec(_H), _dp_spec, _full_spec((1, _H)),
                  _full_spec((_H, _H)), _full_spec((1, _H)),
                  _full_spec((_H, 1)), _full_spec((1, 1)),
                  _full_spec((_H, _H)), _full_spec((1, _H)),
                  _full_spec((_H, 1)), _full_spec((1, 1))],
        out_specs=[_row_spec(1),
                   pl.BlockSpec((1, _H), lambda i: (0, 0)),
                   pl.BlockSpec((1, _H), lambda i: (0, 0))],
        out_shape=[jax.ShapeDtypeStruct((_N, 1), jnp.float32),
                   jax.ShapeDtypeStruct((1, _H), jnp.float32),
                   jax.ShapeDtypeStruct((1, _H), jnp.float32)],
    )(a2, g2, dp, gcn_b2.reshape(1, _H),
      lp_w1, lp_b1.reshape(1, _H), lp_w2, lp_b2.reshape(1, 1),
      v_w1, v_b1.reshape(1, _H), v_w2, v_b2.reshape(1, 1))

    return probs[:, 0], sv[0, :1]
